# baseline (speedup 1.0000x reference)
import functools

import jax
import jax.numpy as jnp
from jax import lax
from jax.experimental import pallas as pl
from jax.experimental.pallas import tpu as pltpu
from jax.experimental.pallas import tpu_sc as plsc

N_NODES = 10000
D = 128
DH = D // 2
NC, NS = 2, 16
NW = NC * NS
CH = 128
N_PAD = 10240
RPT = N_PAD // NS
NB = 6


def _mesh():
    return plsc.VectorSubcoreMesh(
        core_axis_name="c", subcore_axis_name="s", num_cores=NC, num_subcores=NS
    )


@functools.lru_cache(maxsize=None)
def _make_deg_kernel(nchd):
    @functools.partial(
        pl.kernel,
        out_type=jax.ShapeDtypeStruct((NC, N_PAD), jnp.float32),
        mesh=_mesh(),
        scratch_types=[
            pltpu.VMEM((nchd, CH), jnp.int32),
            pltpu.VMEM((CH,), jnp.float32),
            pltpu.VMEM((RPT,), jnp.float32),
            pltpu.VMEM_SHARED((N_PAD,), jnp.float32),
            pltpu.SemaphoreType.DMA,
        ],
    )
    def deg_kernel(dst_hbm, out_hbm, didx, ones_v, zer_v, deg_sh, dsem):
        cid = lax.axis_index("c")
        sid = lax.axis_index("s")
        tid = cid * NS + sid
        pltpu.sync_copy(dst_hbm.at[tid], didx)
        for i in range(CH // 16):
            ones_v[pl.ds(i * 16, 16)] = jnp.ones((16,), jnp.float32)
        for i in range(RPT // 16):
            zer_v[pl.ds(i * 16, 16)] = jnp.zeros((16,), jnp.float32)
        pltpu.sync_copy(zer_v, deg_sh.at[pl.ds(sid * RPT, RPT)])
        plsc.subcore_barrier()
        dd = [
            pltpu.async_copy(ones_v, deg_sh.at[didx.at[c]], dsem, add=True)
            for c in range(nchd)
        ]
        for x in dd:
            x.wait()
        plsc.subcore_barrier()
        pltpu.sync_copy(
            deg_sh.at[pl.ds(sid * RPT, RPT)],
            out_hbm.at[cid, pl.ds(sid * RPT, RPT)],
        )

    return deg_kernel


@functools.lru_cache(maxsize=None)
def _make_layer_kernel(nch):
    scratch = (
        [pltpu.VMEM((nch, CH), jnp.int32)] * 2
        + [pltpu.VMEM((CH, DH), jnp.float32) for _ in range(NB)]
        + [pltpu.SemaphoreType.DMA for _ in range(2 * NB)]
        + [pltpu.VMEM_SHARED((N_PAD, DH), jnp.float32)]
    )

    @functools.partial(
        pl.kernel,
        out_type=jax.ShapeDtypeStruct((NC * N_PAD, DH), jnp.float32),
        mesh=_mesh(),
        scratch_types=scratch,
        compiler_params=pltpu.CompilerParams(use_tc_tiling_on_sc=False),
    )
    def layer_kernel(ylo_hbm, yhi_hbm, src_hbm, dst_hbm, out_hbm, sidx, didx, *rest):
        bufs = rest[0:NB]
        gsems = rest[NB : 2 * NB]
        ssems = rest[2 * NB : 3 * NB]
        acc_sh = rest[3 * NB]
        cid = lax.axis_index("c")
        sid = lax.axis_index("s")

        b0 = bufs[0]

        def zbody(i, carry):
            b0[i // (DH // 16), pl.ds((i % (DH // 16)) * 16, 16)] = jnp.zeros(
                (16,), jnp.float32
            )
            return carry

        lax.fori_loop(0, CH * (DH // 16), zbody, 0)
        zd = [
            pltpu.async_copy(
                b0, acc_sh.at[pl.ds(sid * RPT + k * CH, CH)], gsems[k % NB]
            )
            for k in range(RPT // CH)
        ]
        pltpu.sync_copy(src_hbm.at[sid], sidx)
        pltpu.sync_copy(dst_hbm.at[sid], didx)
        for z in zd:
            z.wait()
        plsc.subcore_barrier()

        def run(y_hbm):
            gd = [None] * nch
            sd = [None] * nch
            for c in range(min(NB, nch)):
                gd[c] = pltpu.async_copy(
                    y_hbm.at[sidx.at[c]], bufs[c % NB], gsems[c % NB]
                )
            for c in range(nch):
                b = c % NB
                gd[c].wait()
                sd[c] = pltpu.async_copy(
                    bufs[b], acc_sh.at[didx.at[c]], ssems[b], add=True
                )
                nc = c + NB
                if nc < nch:
                    sd[c].wait()
                    gd[nc] = pltpu.async_copy(
                        y_hbm.at[sidx.at[nc]], bufs[b], gsems[b]
                    )
            for c in range(max(0, nch - NB), nch):
                sd[c].wait()

        @pl.when(cid == 0)
        def _():
            run(ylo_hbm)

        @pl.when(cid == 1)
        def _():
            run(yhi_hbm)

        plsc.subcore_barrier()
        wd = [
            pltpu.async_copy(
                acc_sh.at[pl.ds(sid * RPT + k * CH, CH)],
                out_hbm.at[pl.ds(cid * N_PAD + sid * RPT + k * CH, CH)],
                gsems[k % NB],
            )
            for k in range(RPT // CH)
        ]
        for w in wd:
            w.wait()

    return layer_kernel


BR = 2048
_GRID = (-(-N_NODES // BR),)


def _rows(i):
    return (i, 0)


def _full(i):
    return (0, 0)


def _tc1_body(degp_ref, x_ref, w_ref, ylo_ref, yhi_ref, d_ref):
    deg = degp_ref[0] + degp_ref[1] + 1.0
    dcol = jnp.reshape(lax.rsqrt(deg), (BR, 1))
    d_ref[...] = dcol
    xw = jnp.dot(x_ref[...], w_ref[...], preferred_element_type=jnp.float32)
    y = dcol * xw
    ylo_ref[...] = y[:, :DH]
    yhi_ref[...] = y[:, DH:]


_tc1 = pl.pallas_call(
    _tc1_body,
    grid=_GRID,
    in_specs=[
        pl.BlockSpec((NC, BR), lambda i: (0, i)),
        pl.BlockSpec((BR, D), _rows),
        pl.BlockSpec((D, D), _full),
    ],
    out_specs=[
        pl.BlockSpec((BR, DH), _rows),
        pl.BlockSpec((BR, DH), _rows),
        pl.BlockSpec((BR, 1), _rows),
    ],
    out_shape=[
        jax.ShapeDtypeStruct((N_NODES, DH), jnp.float32),
        jax.ShapeDtypeStruct((N_NODES, DH), jnp.float32),
        jax.ShapeDtypeStruct((N_NODES, 1), jnp.float32),
    ],
)


_ACC_SPECS = [
    pl.BlockSpec((BR, DH), _rows),
    pl.BlockSpec((BR, DH), lambda i: (i + N_PAD // BR, 0)),
]


def _agg(ylo_ref, yhi_ref, alo_ref, ahi_ref):
    slo = ylo_ref[...] + alo_ref[...]
    shi = yhi_ref[...] + ahi_ref[...]
    return jnp.concatenate([slo, shi], axis=1)


def _tc2_body(ylo_ref, yhi_ref, alo_ref, ahi_ref, d_ref, b_ref, w_ref,
              olo_ref, ohi_ref):
    h = jnp.maximum(
        d_ref[...] * _agg(ylo_ref, yhi_ref, alo_ref, ahi_ref) + b_ref[...], 0.0
    )
    y2 = d_ref[...] * jnp.dot(h, w_ref[...], preferred_element_type=jnp.float32)
    olo_ref[...] = y2[:, :DH]
    ohi_ref[...] = y2[:, DH:]


_tc2 = pl.pallas_call(
    _tc2_body,
    grid=_GRID,
    in_specs=[
        pl.BlockSpec((BR, DH), _rows),
        pl.BlockSpec((BR, DH), _rows),
        *_ACC_SPECS,
        pl.BlockSpec((BR, 1), _rows),
        pl.BlockSpec((1, D), _full),
        pl.BlockSpec((D, D), _full),
    ],
    out_specs=[
        pl.BlockSpec((BR, DH), _rows),
        pl.BlockSpec((BR, DH), _rows),
    ],
    out_shape=[
        jax.ShapeDtypeStruct((N_NODES, DH), jnp.float32),
        jax.ShapeDtypeStruct((N_NODES, DH), jnp.float32),
    ],
)


def _tc3_body(ylo_ref, yhi_ref, alo_ref, ahi_ref, d_ref, b_ref, w_ref,
              bfc_ref, o_ref):
    h = jnp.maximum(
        d_ref[...] * _agg(ylo_ref, yhi_ref, alo_ref, ahi_ref) + b_ref[...], 0.0
    )
    o_ref[...] = (
        jnp.dot(h, w_ref[...], preferred_element_type=jnp.float32) + bfc_ref[...]
    )


_tc3 = pl.pallas_call(
    _tc3_body,
    grid=_GRID,
    in_specs=[
        pl.BlockSpec((BR, DH), _rows),
        pl.BlockSpec((BR, DH), _rows),
        *_ACC_SPECS,
        pl.BlockSpec((BR, 1), _rows),
        pl.BlockSpec((1, D), _full),
        pl.BlockSpec((D, D), _full),
        pl.BlockSpec((1, D), _full),
    ],
    out_specs=pl.BlockSpec((BR, D), _rows),
    out_shape=jax.ShapeDtypeStruct((N_NODES, D), jnp.float32),
)


def kernel(x, edge_index, W1, b1, W2, b2, Wfc, bfc):
    e = edge_index.shape[1]
    ei = edge_index.astype(jnp.int32)

    nchd = -(-e // (NW * CH))
    epd = NW * nchd * CH
    nch = epd // (NS * CH)
    src = jnp.concatenate([ei[0], jnp.zeros((epd - e,), jnp.int32)])
    dst = jnp.concatenate([ei[1], jnp.full((epd - e,), N_NODES, jnp.int32)])
    dstd3 = dst.reshape(NW, nchd, CH)
    src3 = src.reshape(NS, nch, CH)
    dst3 = dst.reshape(NS, nch, CH)

    deg_k = _make_deg_kernel(nchd)
    layer_k = _make_layer_kernel(nch)

    degp = deg_k(dstd3)
    y1lo, y1hi, d = _tc1(degp, x, W1)
    acc1 = layer_k(y1lo, y1hi, src3, dst3)
    y2lo, y2hi = _tc2(y1lo, y1hi, acc1, acc1, d, b1.reshape(1, D), W2)
    acc2 = layer_k(y2lo, y2hi, src3, dst3)
    return _tc3(
        y2lo, y2hi, acc2, acc2, d, b2.reshape(1, D), Wfc, bfc.reshape(1, D)
    )


# speedup vs baseline: 1.3371x; 1.3371x over previous
import functools

import jax
import jax.numpy as jnp
from jax import lax
from jax.experimental import pallas as pl
from jax.experimental.pallas import tpu as pltpu
from jax.experimental.pallas import tpu_sc as plsc

N_NODES = 10000
D = 128
DH = D // 2
NC, NS = 2, 16
NW = NC * NS
CH = 128
N_PAD = 10240
RPT = N_PAD // NS
NB = 6


def _mesh():
    return plsc.VectorSubcoreMesh(
        core_axis_name="c", subcore_axis_name="s", num_cores=NC, num_subcores=NS
    )


@functools.lru_cache(maxsize=None)
def _make_deg_kernel(nchd):
    @functools.partial(
        pl.kernel,
        out_type=jax.ShapeDtypeStruct((NC, N_PAD), jnp.float32),
        mesh=_mesh(),
        scratch_types=[
            pltpu.VMEM((nchd, CH), jnp.int32),
            pltpu.VMEM((CH,), jnp.float32),
            pltpu.VMEM((RPT,), jnp.float32),
            pltpu.VMEM_SHARED((N_PAD,), jnp.float32),
            pltpu.SemaphoreType.DMA,
        ],
    )
    def deg_kernel(dst_hbm, out_hbm, didx, ones_v, zer_v, deg_sh, dsem):
        cid = lax.axis_index("c")
        sid = lax.axis_index("s")
        tid = cid * NS + sid
        pltpu.sync_copy(dst_hbm.at[tid], didx)
        for i in range(CH // 16):
            ones_v[pl.ds(i * 16, 16)] = jnp.ones((16,), jnp.float32)
        for i in range(RPT // 16):
            zer_v[pl.ds(i * 16, 16)] = jnp.zeros((16,), jnp.float32)
        pltpu.sync_copy(zer_v, deg_sh.at[pl.ds(sid * RPT, RPT)])
        plsc.subcore_barrier()
        dd = [
            pltpu.async_copy(ones_v, deg_sh.at[didx.at[c]], dsem, add=True)
            for c in range(nchd)
        ]
        for x in dd:
            x.wait()
        plsc.subcore_barrier()
        pltpu.sync_copy(
            deg_sh.at[pl.ds(sid * RPT, RPT)],
            out_hbm.at[cid, pl.ds(sid * RPT, RPT)],
        )

    return deg_kernel


@functools.lru_cache(maxsize=None)
def _make_layer_kernel(nch):
    scratch = (
        [pltpu.VMEM((nch, CH), jnp.int32)] * 2
        + [pltpu.VMEM((CH, DH), jnp.float32) for _ in range(NB)]
        + [pltpu.SemaphoreType.DMA for _ in range(2 * NB)]
        + [pltpu.VMEM_SHARED((N_PAD, DH), jnp.float32)]
    )

    @functools.partial(
        pl.kernel,
        out_type=jax.ShapeDtypeStruct((NC * N_PAD, DH), jnp.float32),
        mesh=_mesh(),
        scratch_types=scratch,
        compiler_params=pltpu.CompilerParams(use_tc_tiling_on_sc=False),
    )
    def layer_kernel(ylo_hbm, yhi_hbm, src_hbm, dst_hbm, out_hbm, sidx, didx, *rest):
        bufs = rest[0:NB]
        gsems = rest[NB : 2 * NB]
        ssems = rest[2 * NB : 3 * NB]
        acc_sh = rest[3 * NB]
        cid = lax.axis_index("c")
        sid = lax.axis_index("s")

        b0 = bufs[0]

        def zbody(i, carry):
            b0[i // (DH // 16), pl.ds((i % (DH // 16)) * 16, 16)] = jnp.zeros(
                (16,), jnp.float32
            )
            return carry

        lax.fori_loop(0, CH * (DH // 16), zbody, 0)
        zd = [
            pltpu.async_copy(
                b0, acc_sh.at[pl.ds(sid * RPT + k * CH, CH)], gsems[k % NB]
            )
            for k in range(RPT // CH)
        ]
        pltpu.sync_copy(src_hbm.at[sid], sidx)
        pltpu.sync_copy(dst_hbm.at[sid], didx)
        for z in zd:
            z.wait()
        plsc.subcore_barrier()

        def run(y_hbm):
            gd = [None] * nch
            sd = [None] * nch
            for c in range(min(NB, nch)):
                gd[c] = pltpu.async_copy(
                    y_hbm.at[sidx.at[c]], bufs[c % NB], gsems[c % NB]
                )
            for c in range(nch):
                b = c % NB
                gd[c].wait()
                sd[c] = pltpu.async_copy(
                    bufs[b], acc_sh.at[didx.at[c]], ssems[b], add=True
                )
                nc = c + NB
                if nc < nch:
                    sd[c].wait()
                    gd[nc] = pltpu.async_copy(
                        y_hbm.at[sidx.at[nc]], bufs[b], gsems[b]
                    )
            for c in range(max(0, nch - NB), nch):
                sd[c].wait()

        @pl.when(cid == 0)
        def _():
            run(ylo_hbm)

        @pl.when(cid == 1)
        def _():
            run(yhi_hbm)

        plsc.subcore_barrier()
        wd = [
            pltpu.async_copy(
                acc_sh.at[pl.ds(sid * RPT + k * CH, CH)],
                out_hbm.at[pl.ds(cid * N_PAD + sid * RPT + k * CH, CH)],
                gsems[k % NB],
            )
            for k in range(RPT // CH)
        ]
        for w in wd:
            w.wait()

    return layer_kernel


BR = 2048
_GRID = (-(-N_NODES // BR),)


def _rows(i):
    return (i, 0)


def _full(i):
    return (0, 0)


def _tc1_body(degp_ref, x_ref, w_ref, ylo_ref, yhi_ref, d_ref):
    deg = degp_ref[0] + degp_ref[1] + 1.0
    dcol = jnp.reshape(lax.rsqrt(deg), (BR, 1))
    d_ref[...] = dcol
    xw = jnp.dot(x_ref[...], w_ref[...], preferred_element_type=jnp.float32)
    y = dcol * xw
    ylo_ref[...] = y[:, :DH]
    yhi_ref[...] = y[:, DH:]


_tc1 = pl.pallas_call(
    _tc1_body,
    grid=_GRID,
    in_specs=[
        pl.BlockSpec((NC, BR), lambda i: (0, i)),
        pl.BlockSpec((BR, D), _rows),
        pl.BlockSpec((D, D), _full),
    ],
    out_specs=[
        pl.BlockSpec((BR, DH), _rows),
        pl.BlockSpec((BR, DH), _rows),
        pl.BlockSpec((BR, 1), _rows),
    ],
    out_shape=[
        jax.ShapeDtypeStruct((N_NODES, DH), jnp.float32),
        jax.ShapeDtypeStruct((N_NODES, DH), jnp.float32),
        jax.ShapeDtypeStruct((N_NODES, 1), jnp.float32),
    ],
)


_ACC_SPECS = [
    pl.BlockSpec((BR, DH), _rows),
    pl.BlockSpec((BR, DH), lambda i: (i + N_PAD // BR, 0)),
]


def _agg(ylo_ref, yhi_ref, alo_ref, ahi_ref):
    slo = ylo_ref[...] + alo_ref[...]
    shi = yhi_ref[...] + ahi_ref[...]
    return jnp.concatenate([slo, shi], axis=1)


def _tc2_body(ylo_ref, yhi_ref, alo_ref, ahi_ref, d_ref, b_ref, w_ref,
              olo_ref, ohi_ref):
    h = jnp.maximum(
        d_ref[...] * _agg(ylo_ref, yhi_ref, alo_ref, ahi_ref) + b_ref[...], 0.0
    )
    y2 = d_ref[...] * jnp.dot(h, w_ref[...], preferred_element_type=jnp.float32)
    olo_ref[...] = y2[:, :DH]
    ohi_ref[...] = y2[:, DH:]


_tc2 = pl.pallas_call(
    _tc2_body,
    grid=_GRID,
    in_specs=[
        pl.BlockSpec((BR, DH), _rows),
        pl.BlockSpec((BR, DH), _rows),
        *_ACC_SPECS,
        pl.BlockSpec((BR, 1), _rows),
        pl.BlockSpec((1, D), _full),
        pl.BlockSpec((D, D), _full),
    ],
    out_specs=[
        pl.BlockSpec((BR, DH), _rows),
        pl.BlockSpec((BR, DH), _rows),
    ],
    out_shape=[
        jax.ShapeDtypeStruct((N_NODES, DH), jnp.float32),
        jax.ShapeDtypeStruct((N_NODES, DH), jnp.float32),
    ],
)


def _tc3_body(ylo_ref, yhi_ref, alo_ref, ahi_ref, d_ref, b_ref, w_ref,
              bfc_ref, o_ref):
    h = jnp.maximum(
        d_ref[...] * _agg(ylo_ref, yhi_ref, alo_ref, ahi_ref) + b_ref[...], 0.0
    )
    o_ref[...] = (
        jnp.dot(h, w_ref[...], preferred_element_type=jnp.float32) + bfc_ref[...]
    )


_tc3 = pl.pallas_call(
    _tc3_body,
    grid=_GRID,
    in_specs=[
        pl.BlockSpec((BR, DH), _rows),
        pl.BlockSpec((BR, DH), _rows),
        *_ACC_SPECS,
        pl.BlockSpec((BR, 1), _rows),
        pl.BlockSpec((1, D), _full),
        pl.BlockSpec((D, D), _full),
        pl.BlockSpec((1, D), _full),
    ],
    out_specs=pl.BlockSpec((BR, D), _rows),
    out_shape=jax.ShapeDtypeStruct((N_NODES, D), jnp.float32),
)


def kernel(x, edge_index, W1, b1, W2, b2, Wfc, bfc):
    e = edge_index.shape[1]
    ei = edge_index.astype(jnp.int32)

    nchd = -(-e // (NW * CH))
    epd = NW * nchd * CH
    dstd = jnp.concatenate([ei[1], jnp.full((epd - e,), N_NODES, jnp.int32)])
    dstd3 = dstd.reshape(NW, nchd, CH)
    nch = -(-e // (NS * CH))
    ep = NS * nch * CH
    src = jnp.concatenate([ei[0], jnp.zeros((ep - e,), jnp.int32)])
    dst = jnp.concatenate([ei[1], jnp.full((ep - e,), N_NODES, jnp.int32)])
    src3 = src.reshape(NS, nch, CH)
    dst3 = dst.reshape(NS, nch, CH)

    deg_k = _make_deg_kernel(nchd)
    layer_k = _make_layer_kernel(nch)

    degp = deg_k(dstd3)
    y1lo, y1hi, d = _tc1(degp, x, W1)
    acc1 = layer_k(y1lo, y1hi, src3, dst3)
    y2lo, y2hi = _tc2(y1lo, y1hi, acc1, acc1, d, b1.reshape(1, D), W2)
    acc2 = layer_k(y2lo, y2hi, src3, dst3)
    return _tc3(
        y2lo, y2hi, acc2, acc2, d, b2.reshape(1, D), Wfc, bfc.reshape(1, D)
    )


# speedup vs baseline: 1.3371x; 1.0000x over previous
import functools

import jax
import jax.numpy as jnp
from jax import lax
from jax.experimental import pallas as pl
from jax.experimental.pallas import tpu as pltpu
from jax.experimental.pallas import tpu_sc as plsc

N_NODES = 10000
D = 128
DH = D // 2
NC, NS = 2, 16
NW = NC * NS
CH = 128
N_PAD = 10240
RPT = N_PAD // NS
NB = 6


def _mesh():
    return plsc.VectorSubcoreMesh(
        core_axis_name="c", subcore_axis_name="s", num_cores=NC, num_subcores=NS
    )


@functools.lru_cache(maxsize=None)
def _make_deg_kernel(nchd):
    @functools.partial(
        pl.kernel,
        out_type=jax.ShapeDtypeStruct((NC, N_PAD), jnp.float32),
        mesh=_mesh(),
        scratch_types=[
            pltpu.VMEM((nchd, CH), jnp.int32),
            pltpu.VMEM((CH,), jnp.float32),
            pltpu.VMEM((RPT,), jnp.float32),
            pltpu.VMEM_SHARED((N_PAD,), jnp.float32),
            pltpu.SemaphoreType.DMA,
        ],
    )
    def deg_kernel(dst_hbm, out_hbm, didx, ones_v, zer_v, deg_sh, dsem):
        cid = lax.axis_index("c")
        sid = lax.axis_index("s")
        tid = cid * NS + sid
        pltpu.sync_copy(dst_hbm.at[tid], didx)
        for i in range(CH // 16):
            ones_v[pl.ds(i * 16, 16)] = jnp.ones((16,), jnp.float32)
        for i in range(RPT // 16):
            zer_v[pl.ds(i * 16, 16)] = jnp.zeros((16,), jnp.float32)
        pltpu.sync_copy(zer_v, deg_sh.at[pl.ds(sid * RPT, RPT)])
        plsc.subcore_barrier()
        dd = [
            pltpu.async_copy(ones_v, deg_sh.at[didx.at[c]], dsem, add=True)
            for c in range(nchd)
        ]
        for x in dd:
            x.wait()
        plsc.subcore_barrier()
        pltpu.sync_copy(
            deg_sh.at[pl.ds(sid * RPT, RPT)],
            out_hbm.at[cid, pl.ds(sid * RPT, RPT)],
        )

    return deg_kernel


@functools.lru_cache(maxsize=None)
def _make_layer_kernel(nch):
    scratch = (
        [pltpu.VMEM_SHARED((N_PAD, DH), jnp.float32)]
        + [pltpu.VMEM((nch, CH), jnp.int32)] * 2
        + [pltpu.VMEM((CH, DH), jnp.float32) for _ in range(NB)]
        + [pltpu.SemaphoreType.DMA for _ in range(2 * NB)]
    )

    @functools.partial(
        pl.kernel,
        out_type=jax.ShapeDtypeStruct((NC * N_PAD, DH), jnp.float32),
        mesh=_mesh(),
        scratch_types=scratch,
        compiler_params=pltpu.CompilerParams(use_tc_tiling_on_sc=False),
    )
    def layer_kernel(ylo_hbm, yhi_hbm, src_hbm, dst_hbm, out_hbm, acc_sh, sidx,
                     didx, *rest):
        bufs = rest[0:NB]
        gsems = rest[NB : 2 * NB]
        ssems = rest[2 * NB : 3 * NB]
        cid = lax.axis_index("c")
        sid = lax.axis_index("s")

        b0 = bufs[0]

        def zbody(i, carry):
            b0[i // (DH // 16), pl.ds((i % (DH // 16)) * 16, 16)] = jnp.zeros(
                (16,), jnp.float32
            )
            return carry

        lax.fori_loop(0, CH * (DH // 16), zbody, 0)
        zd = [
            pltpu.async_copy(
                b0, acc_sh.at[pl.ds(sid * RPT + k * CH, CH)], gsems[k % NB]
            )
            for k in range(RPT // CH)
        ]
        pltpu.sync_copy(src_hbm.at[sid], sidx)
        pltpu.sync_copy(dst_hbm.at[sid], didx)
        for z in zd:
            z.wait()
        plsc.subcore_barrier()

        def run(y_hbm):
            gd = [None] * nch
            sd = [None] * nch
            for c in range(min(NB, nch)):
                gd[c] = pltpu.async_copy(
                    y_hbm.at[sidx.at[c]], bufs[c % NB], gsems[c % NB]
                )
            for c in range(nch):
                b = c % NB
                gd[c].wait()
                sd[c] = pltpu.async_copy(
                    bufs[b], acc_sh.at[didx.at[c]], ssems[b], add=True
                )
                nc = c + NB
                if nc < nch:
                    sd[c].wait()
                    gd[nc] = pltpu.async_copy(
                        y_hbm.at[sidx.at[nc]], bufs[b], gsems[b]
                    )
            for c in range(max(0, nch - NB), nch):
                sd[c].wait()

        @pl.when(cid == 0)
        def _():
            run(ylo_hbm)

        @pl.when(cid == 1)
        def _():
            run(yhi_hbm)

        plsc.subcore_barrier()
        wd = [
            pltpu.async_copy(
                acc_sh.at[pl.ds(sid * RPT + k * CH, CH)],
                out_hbm.at[pl.ds(cid * N_PAD + sid * RPT + k * CH, CH)],
                gsems[k % NB],
            )
            for k in range(RPT // CH)
        ]
        for w in wd:
            w.wait()

    return layer_kernel


BR = 2048
_GRID = (-(-N_NODES // BR),)


def _rows(i):
    return (i, 0)


def _full(i):
    return (0, 0)


def _tc1_body(degp_ref, x_ref, w_ref, ylo_ref, yhi_ref, d_ref):
    deg = degp_ref[0] + degp_ref[1] + 1.0
    dcol = jnp.reshape(lax.rsqrt(deg), (BR, 1))
    d_ref[...] = dcol
    xw = jnp.dot(x_ref[...], w_ref[...], preferred_element_type=jnp.float32)
    y = dcol * xw
    ylo_ref[...] = y[:, :DH]
    yhi_ref[...] = y[:, DH:]


_tc1 = pl.pallas_call(
    _tc1_body,
    grid=_GRID,
    in_specs=[
        pl.BlockSpec((NC, BR), lambda i: (0, i)),
        pl.BlockSpec((BR, D), _rows),
        pl.BlockSpec((D, D), _full),
    ],
    out_specs=[
        pl.BlockSpec((BR, DH), _rows),
        pl.BlockSpec((BR, DH), _rows),
        pl.BlockSpec((BR, 1), _rows),
    ],
    out_shape=[
        jax.ShapeDtypeStruct((N_NODES, DH), jnp.float32),
        jax.ShapeDtypeStruct((N_NODES, DH), jnp.float32),
        jax.ShapeDtypeStruct((N_NODES, 1), jnp.float32),
    ],
)


_ACC_SPECS = [
    pl.BlockSpec((BR, DH), _rows),
    pl.BlockSpec((BR, DH), lambda i: (i + N_PAD // BR, 0)),
]


def _agg(ylo_ref, yhi_ref, alo_ref, ahi_ref):
    slo = ylo_ref[...] + alo_ref[...]
    shi = yhi_ref[...] + ahi_ref[...]
    return jnp.concatenate([slo, shi], axis=1)


def _tc2_body(ylo_ref, yhi_ref, alo_ref, ahi_ref, d_ref, b_ref, w_ref,
              olo_ref, ohi_ref):
    h = jnp.maximum(
        d_ref[...] * _agg(ylo_ref, yhi_ref, alo_ref, ahi_ref) + b_ref[...], 0.0
    )
    y2 = d_ref[...] * jnp.dot(h, w_ref[...], preferred_element_type=jnp.float32)
    olo_ref[...] = y2[:, :DH]
    ohi_ref[...] = y2[:, DH:]


_tc2 = pl.pallas_call(
    _tc2_body,
    grid=_GRID,
    in_specs=[
        pl.BlockSpec((BR, DH), _rows),
        pl.BlockSpec((BR, DH), _rows),
        *_ACC_SPECS,
        pl.BlockSpec((BR, 1), _rows),
        pl.BlockSpec((1, D), _full),
        pl.BlockSpec((D, D), _full),
    ],
    out_specs=[
        pl.BlockSpec((BR, DH), _rows),
        pl.BlockSpec((BR, DH), _rows),
    ],
    out_shape=[
        jax.ShapeDtypeStruct((N_NODES, DH), jnp.float32),
        jax.ShapeDtypeStruct((N_NODES, DH), jnp.float32),
    ],
)


def _tc3_body(ylo_ref, yhi_ref, alo_ref, ahi_ref, d_ref, b_ref, w_ref,
              bfc_ref, o_ref):
    h = jnp.maximum(
        d_ref[...] * _agg(ylo_ref, yhi_ref, alo_ref, ahi_ref) + b_ref[...], 0.0
    )
    o_ref[...] = (
        jnp.dot(h, w_ref[...], preferred_element_type=jnp.float32) + bfc_ref[...]
    )


_tc3 = pl.pallas_call(
    _tc3_body,
    grid=_GRID,
    in_specs=[
        pl.BlockSpec((BR, DH), _rows),
        pl.BlockSpec((BR, DH), _rows),
        *_ACC_SPECS,
        pl.BlockSpec((BR, 1), _rows),
        pl.BlockSpec((1, D), _full),
        pl.BlockSpec((D, D), _full),
        pl.BlockSpec((1, D), _full),
    ],
    out_specs=pl.BlockSpec((BR, D), _rows),
    out_shape=jax.ShapeDtypeStruct((N_NODES, D), jnp.float32),
)


def kernel(x, edge_index, W1, b1, W2, b2, Wfc, bfc):
    e = edge_index.shape[1]
    ei = edge_index.astype(jnp.int32)

    nchd = -(-e // (NW * CH))
    epd = NW * nchd * CH
    dstd = jnp.concatenate([ei[1], jnp.full((epd - e,), N_NODES, jnp.int32)])
    dstd3 = dstd.reshape(NW, nchd, CH)
    nch = -(-e // (NS * CH))
    ep = NS * nch * CH
    src = jnp.concatenate([ei[0], jnp.zeros((ep - e,), jnp.int32)])
    dst = jnp.concatenate([ei[1], jnp.full((ep - e,), N_NODES, jnp.int32)])
    src3 = src.reshape(NS, nch, CH)
    dst3 = dst.reshape(NS, nch, CH)

    deg_k = _make_deg_kernel(nchd)
    layer_k = _make_layer_kernel(nch)

    degp = deg_k(dstd3)
    y1lo, y1hi, d = _tc1(degp, x, W1)
    acc1 = layer_k(y1lo, y1hi, src3, dst3)
    y2lo, y2hi = _tc2(y1lo, y1hi, acc1, acc1, d, b1.reshape(1, D), W2)
    acc2 = layer_k(y2lo, y2hi, src3, dst3)
    return _tc3(
        y2lo, y2hi, acc2, acc2, d, b2.reshape(1, D), Wfc, bfc.reshape(1, D)
    )


# speedup vs baseline: 1.3426x; 1.0041x over previous
import functools

import jax
import jax.numpy as jnp
from jax import lax
from jax.experimental import pallas as pl
from jax.experimental.pallas import tpu as pltpu
from jax.experimental.pallas import tpu_sc as plsc

N_NODES = 10000
D = 128
DH = D // 2
NC, NS = 2, 16
NW = NC * NS
CH = 128
N_PAD = 10240
RPT = N_PAD // NS
NB = 6


def _mesh():
    return plsc.VectorSubcoreMesh(
        core_axis_name="c", subcore_axis_name="s", num_cores=NC, num_subcores=NS
    )


@functools.lru_cache(maxsize=None)
def _make_deg_kernel(nchd):
    @functools.partial(
        pl.kernel,
        out_type=jax.ShapeDtypeStruct((NC, N_PAD), jnp.float32),
        mesh=_mesh(),
        scratch_types=[
            pltpu.VMEM((nchd, CH), jnp.int32),
            pltpu.VMEM((CH,), jnp.float32),
            pltpu.VMEM((RPT,), jnp.float32),
            pltpu.VMEM_SHARED((N_PAD,), jnp.float32),
            pltpu.SemaphoreType.DMA,
        ],
    )
    def deg_kernel(dst_hbm, out_hbm, didx, ones_v, zer_v, deg_sh, dsem):
        cid = lax.axis_index("c")
        sid = lax.axis_index("s")
        tid = cid * NS + sid
        pltpu.sync_copy(dst_hbm.at[tid], didx)
        for i in range(CH // 16):
            ones_v[pl.ds(i * 16, 16)] = jnp.ones((16,), jnp.float32)
        for i in range(RPT // 16):
            zer_v[pl.ds(i * 16, 16)] = jnp.zeros((16,), jnp.float32)
        pltpu.sync_copy(zer_v, deg_sh.at[pl.ds(sid * RPT, RPT)])
        plsc.subcore_barrier()
        dd = [
            pltpu.async_copy(ones_v, deg_sh.at[didx.at[c]], dsem, add=True)
            for c in range(nchd)
        ]
        for x in dd:
            x.wait()
        plsc.subcore_barrier()
        pltpu.sync_copy(
            deg_sh.at[pl.ds(sid * RPT, RPT)],
            out_hbm.at[cid, pl.ds(sid * RPT, RPT)],
        )

    return deg_kernel


@functools.lru_cache(maxsize=None)
def _make_layer_kernel(nch):
    scratch = (
        [pltpu.VMEM_SHARED((N_PAD, DH), jnp.float32)]
        + [pltpu.VMEM((nch, CH), jnp.int32)] * 2
        + [pltpu.VMEM((CH, DH), jnp.float32) for _ in range(NB)]
        + [pltpu.SemaphoreType.DMA for _ in range(2 * NB)]
    )

    @functools.partial(
        pl.kernel,
        out_type=jax.ShapeDtypeStruct((NC * N_PAD, DH), jnp.float32),
        mesh=_mesh(),
        scratch_types=scratch,
        compiler_params=pltpu.CompilerParams(use_tc_tiling_on_sc=False),
    )
    def layer_kernel(ylo_hbm, yhi_hbm, src_hbm, dst_hbm, out_hbm, acc_sh, sidx,
                     didx, *rest):
        bufs = rest[0:NB]
        gsems = rest[NB : 2 * NB]
        ssems = rest[2 * NB : 3 * NB]
        cid = lax.axis_index("c")
        sid = lax.axis_index("s")

        b0 = bufs[0]

        def zbody(i, carry):
            b0[i // (DH // 16), pl.ds((i % (DH // 16)) * 16, 16)] = jnp.zeros(
                (16,), jnp.float32
            )
            return carry

        lax.fori_loop(0, CH * (DH // 16), zbody, 0)
        zd = [
            pltpu.async_copy(
                b0, acc_sh.at[pl.ds(sid * RPT + k * CH, CH)], gsems[k % NB]
            )
            for k in range(RPT // CH)
        ]
        pltpu.sync_copy(src_hbm.at[sid], sidx)
        pltpu.sync_copy(dst_hbm.at[sid], didx)
        for z in zd:
            z.wait()
        plsc.subcore_barrier()

        def run(y_hbm):
            gd = [None] * nch
            sd = [None] * nch
            for c in range(min(NB, nch)):
                gd[c] = pltpu.async_copy(
                    y_hbm.at[sidx.at[c]], bufs[c % NB], gsems[c % NB]
                )
            waited = [False] * nch
            for c in range(nch):
                b = c % NB
                gd[c].wait()
                sd[c] = pltpu.async_copy(
                    bufs[b], acc_sh.at[didx.at[c]], ssems[b], add=True
                )
                j = c - 1
                if j >= 0 and j + NB < nch:
                    sd[j].wait()
                    waited[j] = True
                    gd[j + NB] = pltpu.async_copy(
                        y_hbm.at[sidx.at[j + NB]], bufs[j % NB], gsems[j % NB]
                    )
            for c in range(nch):
                if not waited[c]:
                    sd[c].wait()

        @pl.when(cid == 0)
        def _():
            run(ylo_hbm)

        @pl.when(cid == 1)
        def _():
            run(yhi_hbm)

        plsc.subcore_barrier()
        wd = [
            pltpu.async_copy(
                acc_sh.at[pl.ds(sid * RPT + k * CH, CH)],
                out_hbm.at[pl.ds(cid * N_PAD + sid * RPT + k * CH, CH)],
                gsems[k % NB],
            )
            for k in range(RPT // CH)
        ]
        for w in wd:
            w.wait()

    return layer_kernel


BR = 2048
_GRID = (-(-N_NODES // BR),)


def _rows(i):
    return (i, 0)


def _full(i):
    return (0, 0)


def _tc1_body(degp_ref, x_ref, w_ref, ylo_ref, yhi_ref, d_ref):
    deg = degp_ref[0] + degp_ref[1] + 1.0
    dcol = jnp.reshape(lax.rsqrt(deg), (BR, 1))
    d_ref[...] = dcol
    xw = jnp.dot(x_ref[...], w_ref[...], preferred_element_type=jnp.float32)
    y = dcol * xw
    ylo_ref[...] = y[:, :DH]
    yhi_ref[...] = y[:, DH:]


_tc1 = pl.pallas_call(
    _tc1_body,
    grid=_GRID,
    in_specs=[
        pl.BlockSpec((NC, BR), lambda i: (0, i)),
        pl.BlockSpec((BR, D), _rows),
        pl.BlockSpec((D, D), _full),
    ],
    out_specs=[
        pl.BlockSpec((BR, DH), _rows),
        pl.BlockSpec((BR, DH), _rows),
        pl.BlockSpec((BR, 1), _rows),
    ],
    out_shape=[
        jax.ShapeDtypeStruct((N_NODES, DH), jnp.float32),
        jax.ShapeDtypeStruct((N_NODES, DH), jnp.float32),
        jax.ShapeDtypeStruct((N_NODES, 1), jnp.float32),
    ],
)


_ACC_SPECS = [
    pl.BlockSpec((BR, DH), _rows),
    pl.BlockSpec((BR, DH), lambda i: (i + N_PAD // BR, 0)),
]


def _agg(ylo_ref, yhi_ref, alo_ref, ahi_ref):
    slo = ylo_ref[...] + alo_ref[...]
    shi = yhi_ref[...] + ahi_ref[...]
    return jnp.concatenate([slo, shi], axis=1)


def _tc2_body(ylo_ref, yhi_ref, alo_ref, ahi_ref, d_ref, b_ref, w_ref,
              olo_ref, ohi_ref):
    h = jnp.maximum(
        d_ref[...] * _agg(ylo_ref, yhi_ref, alo_ref, ahi_ref) + b_ref[...], 0.0
    )
    y2 = d_ref[...] * jnp.dot(h, w_ref[...], preferred_element_type=jnp.float32)
    olo_ref[...] = y2[:, :DH]
    ohi_ref[...] = y2[:, DH:]


_tc2 = pl.pallas_call(
    _tc2_body,
    grid=_GRID,
    in_specs=[
        pl.BlockSpec((BR, DH), _rows),
        pl.BlockSpec((BR, DH), _rows),
        *_ACC_SPECS,
        pl.BlockSpec((BR, 1), _rows),
        pl.BlockSpec((1, D), _full),
        pl.BlockSpec((D, D), _full),
    ],
    out_specs=[
        pl.BlockSpec((BR, DH), _rows),
        pl.BlockSpec((BR, DH), _rows),
    ],
    out_shape=[
        jax.ShapeDtypeStruct((N_NODES, DH), jnp.float32),
        jax.ShapeDtypeStruct((N_NODES, DH), jnp.float32),
    ],
)


def _tc3_body(ylo_ref, yhi_ref, alo_ref, ahi_ref, d_ref, b_ref, w_ref,
              bfc_ref, o_ref):
    h = jnp.maximum(
        d_ref[...] * _agg(ylo_ref, yhi_ref, alo_ref, ahi_ref) + b_ref[...], 0.0
    )
    o_ref[...] = (
        jnp.dot(h, w_ref[...], preferred_element_type=jnp.float32) + bfc_ref[...]
    )


_tc3 = pl.pallas_call(
    _tc3_body,
    grid=_GRID,
    in_specs=[
        pl.BlockSpec((BR, DH), _rows),
        pl.BlockSpec((BR, DH), _rows),
        *_ACC_SPECS,
        pl.BlockSpec((BR, 1), _rows),
        pl.BlockSpec((1, D), _full),
        pl.BlockSpec((D, D), _full),
        pl.BlockSpec((1, D), _full),
    ],
    out_specs=pl.BlockSpec((BR, D), _rows),
    out_shape=jax.ShapeDtypeStruct((N_NODES, D), jnp.float32),
)


def kernel(x, edge_index, W1, b1, W2, b2, Wfc, bfc):
    e = edge_index.shape[1]
    ei = edge_index.astype(jnp.int32)

    nchd = -(-e // (NW * CH))
    epd = NW * nchd * CH
    dstd = jnp.concatenate([ei[1], jnp.full((epd - e,), N_NODES, jnp.int32)])
    dstd3 = dstd.reshape(NW, nchd, CH)
    nch = -(-e // (NS * CH))
    ep = NS * nch * CH
    src = jnp.concatenate([ei[0], jnp.zeros((ep - e,), jnp.int32)])
    dst = jnp.concatenate([ei[1], jnp.full((ep - e,), N_NODES, jnp.int32)])
    src3 = src.reshape(NS, nch, CH)
    dst3 = dst.reshape(NS, nch, CH)

    deg_k = _make_deg_kernel(nchd)
    layer_k = _make_layer_kernel(nch)

    degp = deg_k(dstd3)
    y1lo, y1hi, d = _tc1(degp, x, W1)
    acc1 = layer_k(y1lo, y1hi, src3, dst3)
    y2lo, y2hi = _tc2(y1lo, y1hi, acc1, acc1, d, b1.reshape(1, D), W2)
    acc2 = layer_k(y2lo, y2hi, src3, dst3)
    return _tc3(
        y2lo, y2hi, acc2, acc2, d, b2.reshape(1, D), Wfc, bfc.reshape(1, D)
    )
